# baseline (speedup 1.0000x reference)
import functools

import jax
import jax.numpy as jnp
from jax import lax
from jax.experimental import pallas as pl
from jax.experimental.pallas import tpu as pltpu

B, H, W = 32, 14, 14
DIM, HEADS, MLP_DIM = 384, 12, 1536
E, TOPK = 8, 2
HEAD_DIM = DIM // HEADS
N = H * W


def _attn_body(x_ref, g1_ref, be1_ref, Wq_ref, bq_ref, Wk_ref, bk_ref,
               Wv_ref, bv_ref, Wo_ref, bo_ref, g2_ref, be2_ref, Wg_ref,
               bg_ref, xnew_ref, nx_ref, ap_ref, ti_ref, tp_ref):
    xb = x_ref[0]

    mu = jnp.mean(xb, axis=-1, keepdims=True)
    var = jnp.mean((xb - mu) ** 2, axis=-1, keepdims=True)
    n1 = (xb - mu) * lax.rsqrt(var + 1e-5) * g1_ref[0] + be1_ref[0]

    q = jnp.dot(n1, Wq_ref[...], preferred_element_type=jnp.float32) + bq_ref[0]
    k = jnp.dot(n1, Wk_ref[...], preferred_element_type=jnp.float32) + bk_ref[0]
    v = jnp.dot(n1, Wv_ref[...], preferred_element_type=jnp.float32) + bv_ref[0]

    ri = lax.broadcasted_iota(jnp.int32, (N, N), 0) // W
    ci = lax.broadcasted_iota(jnp.int32, (N, N), 1) // W
    row_mask = ri == ci

    outs = []
    for h in range(HEADS):
        sl = slice(h * HEAD_DIM, (h + 1) * HEAD_DIM)
        s = lax.dot_general(q[:, sl], k[:, sl], (((1,), (1,)), ((), ())),
                            preferred_element_type=jnp.float32)
        s = s / (HEAD_DIM ** 0.5)
        s = jnp.where(row_mask, s, -1e30)
        m = jnp.max(s, axis=-1, keepdims=True)
        e = jnp.exp(s - m)
        p = e / jnp.sum(e, axis=-1, keepdims=True)
        outs.append(jnp.dot(p, v[:, sl], preferred_element_type=jnp.float32))
    o = jnp.concatenate(outs, axis=1)
    o = jnp.dot(o, Wo_ref[...], preferred_element_type=jnp.float32) + bo_ref[0]
    xn = xb + o
    xnew_ref[0] = xn

    mu2 = jnp.mean(xn, axis=-1, keepdims=True)
    var2 = jnp.mean((xn - mu2) ** 2, axis=-1, keepdims=True)
    nx = (xn - mu2) * lax.rsqrt(var2 + 1e-5) * g2_ref[0] + be2_ref[0]
    nx_ref[0] = nx

    pooled = jnp.mean(nx, axis=0, keepdims=True)
    logits = jnp.dot(pooled, Wg_ref[...],
                     preferred_element_type=jnp.float32) + bg_ref[0]
    lm = jnp.max(logits, axis=-1, keepdims=True)
    ex = jnp.exp(logits - lm)
    probs = ex / jnp.sum(ex, axis=-1, keepdims=True)
    ap_ref[0] = probs

    col = lax.broadcasted_iota(jnp.int32, (1, E), 1)
    m1 = jnp.max(probs, axis=-1, keepdims=True)
    i1 = jnp.min(jnp.where(probs == m1, col, E), axis=-1, keepdims=True)
    pm = jnp.where(col == i1, -1.0, probs)
    m2 = jnp.max(pm, axis=-1, keepdims=True)
    i2 = jnp.min(jnp.where(pm == m2, col, E), axis=-1, keepdims=True)
    den = m1 + m2 + 1e-8
    tp_ref[0] = jnp.concatenate([m1 / den, m2 / den], axis=1)
    ti_ref[0] = jnp.concatenate([i1, i2], axis=1)


def _moe_body(ti_ref, tp_ref, nx_ref, xnew_ref, W1_ref, b1_ref, W2_ref,
              b2_ref, out_ref):
    i = pl.program_id(0)
    j = pl.program_id(1)
    w = tp_ref[i * TOPK + j]
    h = jnp.dot(nx_ref[0].astype(jnp.bfloat16), W1_ref[0],
                preferred_element_type=jnp.float32) + b1_ref[0]
    h = 0.5 * h * (1.0 + lax.erf(h * (2.0 ** -0.5)))
    eo = jnp.dot(h.astype(jnp.bfloat16), W2_ref[0],
                 preferred_element_type=jnp.float32) + b2_ref[0]

    @pl.when(j == 0)
    def _():
        out_ref[0] = xnew_ref[0] + w * eo

    @pl.when(j == 1)
    def _():
        out_ref[0] = out_ref[0] + w * eo


def kernel(x, g1, be1, Wq, bq, Wk, bk, Wv, bv, Wo, bo, g2, be2, Wg, bg,
           W1, b1, W2, b2):
    xf = x.reshape(B, N, DIM)
    r2 = lambda a: a.reshape(1, -1)

    const2 = lambda shape: pl.BlockSpec(shape, lambda b: (0, 0))
    attn_out = pl.pallas_call(
        _attn_body,
        grid=(B,),
        in_specs=[
            pl.BlockSpec((1, N, DIM), lambda b: (b, 0, 0)),
            const2((1, DIM)), const2((1, DIM)),
            const2((DIM, DIM)), const2((1, DIM)),
            const2((DIM, DIM)), const2((1, DIM)),
            const2((DIM, DIM)), const2((1, DIM)),
            const2((DIM, DIM)), const2((1, DIM)),
            const2((1, DIM)), const2((1, DIM)),
            const2((DIM, E)), const2((1, E)),
        ],
        out_specs=[
            pl.BlockSpec((1, N, DIM), lambda b: (b, 0, 0)),
            pl.BlockSpec((1, N, DIM), lambda b: (b, 0, 0)),
            pl.BlockSpec((1, 1, E), lambda b: (b, 0, 0)),
            pl.BlockSpec((1, 1, TOPK), lambda b: (b, 0, 0)),
            pl.BlockSpec((1, 1, TOPK), lambda b: (b, 0, 0)),
        ],
        out_shape=[
            jax.ShapeDtypeStruct((B, N, DIM), jnp.float32),
            jax.ShapeDtypeStruct((B, N, DIM), jnp.float32),
            jax.ShapeDtypeStruct((B, 1, E), jnp.float32),
            jax.ShapeDtypeStruct((B, 1, TOPK), jnp.int32),
            jax.ShapeDtypeStruct((B, 1, TOPK), jnp.float32),
        ],
        compiler_params=pltpu.CompilerParams(
            dimension_semantics=("parallel",)),
    )(xf, r2(g1), r2(be1), Wq, r2(bq), Wk, r2(bk), Wv, r2(bv), Wo, r2(bo),
      r2(g2), r2(be2), Wg, r2(bg))

    xnew, nx, ap3, ti3, tp3 = attn_out
    all_probs = ap3.reshape(B, E)
    ti = ti3.reshape(B, TOPK)
    ti_flat = ti.reshape(B * TOPK)
    tp_flat = tp3.reshape(B * TOPK)

    grid_spec = pltpu.PrefetchScalarGridSpec(
        num_scalar_prefetch=2,
        grid=(B, TOPK),
        in_specs=[
            pl.BlockSpec((1, N, DIM), lambda i, j, ti_s, tp_s: (i, 0, 0)),
            pl.BlockSpec((1, N, DIM), lambda i, j, ti_s, tp_s: (i, 0, 0)),
            pl.BlockSpec((1, DIM, MLP_DIM),
                         lambda i, j, ti_s, tp_s: (ti_s[i * TOPK + j], 0, 0)),
            pl.BlockSpec((1, 1, MLP_DIM),
                         lambda i, j, ti_s, tp_s: (ti_s[i * TOPK + j], 0, 0)),
            pl.BlockSpec((1, MLP_DIM, DIM),
                         lambda i, j, ti_s, tp_s: (ti_s[i * TOPK + j], 0, 0)),
            pl.BlockSpec((1, 1, DIM),
                         lambda i, j, ti_s, tp_s: (ti_s[i * TOPK + j], 0, 0)),
        ],
        out_specs=pl.BlockSpec((1, N, DIM), lambda i, j, ti_s, tp_s: (i, 0, 0)),
    )
    out = pl.pallas_call(
        _moe_body,
        grid_spec=grid_spec,
        out_shape=jax.ShapeDtypeStruct((B, N, DIM), jnp.float32),
        compiler_params=pltpu.CompilerParams(
            dimension_semantics=("parallel", "arbitrary")),
    )(ti_flat, tp_flat, nx, xnew, W1.astype(jnp.bfloat16),
      b1.reshape(E, 1, MLP_DIM), W2.astype(jnp.bfloat16),
      b2.reshape(E, 1, DIM))

    return (out.reshape(B, H, W, DIM), all_probs, ti)


# speedup vs baseline: 1.0641x; 1.0641x over previous
import functools

import jax
import jax.numpy as jnp
from jax import lax
from jax.experimental import pallas as pl
from jax.experimental.pallas import tpu as pltpu

B, H, W = 32, 14, 14
DIM, HEADS, MLP_DIM = 384, 12, 1536
E, TOPK = 8, 2
HEAD_DIM = DIM // HEADS
N = H * W


IMGS = 2


def _attn_body(x_ref, g1_ref, be1_ref, Wq_ref, bq_ref, Wk_ref, bk_ref,
               Wv_ref, bv_ref, Wo_ref, bo_ref, g2_ref, be2_ref, Wg_ref,
               bg_ref, xnew_ref, nx_ref, ap_ref, ti_ref, tp_ref):
    for img in range(IMGS):
        _attn_one(img, x_ref, g1_ref, be1_ref, Wq_ref, bq_ref, Wk_ref,
                  bk_ref, Wv_ref, bv_ref, Wo_ref, bo_ref, g2_ref, be2_ref,
                  Wg_ref, bg_ref, xnew_ref, nx_ref, ap_ref, ti_ref, tp_ref)


def _attn_one(img, x_ref, g1_ref, be1_ref, Wq_ref, bq_ref, Wk_ref, bk_ref,
              Wv_ref, bv_ref, Wo_ref, bo_ref, g2_ref, be2_ref, Wg_ref,
              bg_ref, xnew_ref, nx_ref, ap_ref, ti_ref, tp_ref):
    xb = x_ref[img]

    mu = jnp.mean(xb, axis=-1, keepdims=True)
    var = jnp.mean((xb - mu) ** 2, axis=-1, keepdims=True)
    n1 = (xb - mu) / jnp.sqrt(var + 1e-5) * g1_ref[0] + be1_ref[0]

    q = jnp.dot(n1, Wq_ref[...], preferred_element_type=jnp.float32) + bq_ref[0]
    k = jnp.dot(n1, Wk_ref[...], preferred_element_type=jnp.float32) + bk_ref[0]
    v = jnp.dot(n1, Wv_ref[...], preferred_element_type=jnp.float32) + bv_ref[0]

    ri = lax.broadcasted_iota(jnp.int32, (N, N), 0) // W
    ci = lax.broadcasted_iota(jnp.int32, (N, N), 1) // W
    row_mask = ri == ci

    outs = []
    for h in range(HEADS):
        sl = slice(h * HEAD_DIM, (h + 1) * HEAD_DIM)
        s = lax.dot_general(q[:, sl], k[:, sl], (((1,), (1,)), ((), ())),
                            preferred_element_type=jnp.float32)
        s = s / (HEAD_DIM ** 0.5)
        s = jnp.where(row_mask, s, -1e30)
        m = jnp.max(s, axis=-1, keepdims=True)
        e = jnp.exp(s - m)
        p = e / jnp.sum(e, axis=-1, keepdims=True)
        outs.append(jnp.dot(p, v[:, sl], preferred_element_type=jnp.float32))
    o = jnp.concatenate(outs, axis=1)
    o = jnp.dot(o, Wo_ref[...], preferred_element_type=jnp.float32) + bo_ref[0]
    xn = xb + o
    xnew_ref[img] = xn

    mu2 = jnp.mean(xn, axis=-1, keepdims=True)
    var2 = jnp.mean((xn - mu2) ** 2, axis=-1, keepdims=True)
    nx = (xn - mu2) / jnp.sqrt(var2 + 1e-5) * g2_ref[0] + be2_ref[0]
    nx_ref[img] = nx

    pooled = jnp.mean(nx, axis=0, keepdims=True)
    logits = jnp.dot(pooled, Wg_ref[...],
                     preferred_element_type=jnp.float32) + bg_ref[0]
    lm = jnp.max(logits, axis=-1, keepdims=True)
    ex = jnp.exp(logits - lm)
    probs = ex / jnp.sum(ex, axis=-1, keepdims=True)
    ap_ref[img] = probs

    col = lax.broadcasted_iota(jnp.int32, (1, E), 1)
    m1 = jnp.max(probs, axis=-1, keepdims=True)
    i1 = jnp.min(jnp.where(probs == m1, col, E), axis=-1, keepdims=True)
    pm = jnp.where(col == i1, -1.0, probs)
    m2 = jnp.max(pm, axis=-1, keepdims=True)
    i2 = jnp.min(jnp.where(pm == m2, col, E), axis=-1, keepdims=True)
    den = m1 + m2 + 1e-8
    tp_ref[img] = jnp.concatenate([m1 / den, m2 / den], axis=1)
    ti_ref[img] = jnp.concatenate([i1, i2], axis=1)


def _moe_body(ti_ref, tp_ref, nx_ref, xnew_ref, W1_ref, b1_ref, W2_ref,
              b2_ref, out_ref):
    i = pl.program_id(0)
    j = pl.program_id(1)
    w = tp_ref[i * TOPK + j]
    h = jnp.dot(nx_ref[0].astype(jnp.bfloat16), W1_ref[0],
                preferred_element_type=jnp.float32) + b1_ref[0]
    h = 0.5 * h * (1.0 + lax.erf(h * (2.0 ** -0.5)))
    eo = jnp.dot(h.astype(jnp.bfloat16), W2_ref[0],
                 preferred_element_type=jnp.float32) + b2_ref[0]

    @pl.when(j == 0)
    def _():
        out_ref[0] = xnew_ref[0] + w * eo

    @pl.when(j == 1)
    def _():
        out_ref[0] = out_ref[0] + w * eo


def kernel(x, g1, be1, Wq, bq, Wk, bk, Wv, bv, Wo, bo, g2, be2, Wg, bg,
           W1, b1, W2, b2):
    xf = x.reshape(B, N, DIM)
    r2 = lambda a: a.reshape(1, -1)

    const2 = lambda shape: pl.BlockSpec(shape, lambda b: (0, 0))
    attn_out = pl.pallas_call(
        _attn_body,
        grid=(B // IMGS,),
        in_specs=[
            pl.BlockSpec((IMGS, N, DIM), lambda b: (b, 0, 0)),
            const2((1, DIM)), const2((1, DIM)),
            const2((DIM, DIM)), const2((1, DIM)),
            const2((DIM, DIM)), const2((1, DIM)),
            const2((DIM, DIM)), const2((1, DIM)),
            const2((DIM, DIM)), const2((1, DIM)),
            const2((1, DIM)), const2((1, DIM)),
            const2((DIM, E)), const2((1, E)),
        ],
        out_specs=[
            pl.BlockSpec((IMGS, N, DIM), lambda b: (b, 0, 0)),
            pl.BlockSpec((IMGS, N, DIM), lambda b: (b, 0, 0)),
            pl.BlockSpec((IMGS, 1, E), lambda b: (b, 0, 0)),
            pl.BlockSpec((IMGS, 1, TOPK), lambda b: (b, 0, 0)),
            pl.BlockSpec((IMGS, 1, TOPK), lambda b: (b, 0, 0)),
        ],
        out_shape=[
            jax.ShapeDtypeStruct((B, N, DIM), jnp.float32),
            jax.ShapeDtypeStruct((B, N, DIM), jnp.float32),
            jax.ShapeDtypeStruct((B, 1, E), jnp.float32),
            jax.ShapeDtypeStruct((B, 1, TOPK), jnp.int32),
            jax.ShapeDtypeStruct((B, 1, TOPK), jnp.float32),
        ],
        compiler_params=pltpu.CompilerParams(
            dimension_semantics=("parallel",)),
    )(xf, r2(g1), r2(be1), Wq, r2(bq), Wk, r2(bk), Wv, r2(bv), Wo, r2(bo),
      r2(g2), r2(be2), Wg, r2(bg))

    xnew, nx, ap3, ti3, tp3 = attn_out
    all_probs = ap3.reshape(B, E)
    ti = ti3.reshape(B, TOPK)
    ti_flat = ti.reshape(B * TOPK)
    tp_flat = tp3.reshape(B * TOPK)

    grid_spec = pltpu.PrefetchScalarGridSpec(
        num_scalar_prefetch=2,
        grid=(B, TOPK),
        in_specs=[
            pl.BlockSpec((1, N, DIM), lambda i, j, ti_s, tp_s: (i, 0, 0)),
            pl.BlockSpec((1, N, DIM), lambda i, j, ti_s, tp_s: (i, 0, 0)),
            pl.BlockSpec((1, DIM, MLP_DIM),
                         lambda i, j, ti_s, tp_s: (ti_s[i * TOPK + j], 0, 0)),
            pl.BlockSpec((1, 1, MLP_DIM),
                         lambda i, j, ti_s, tp_s: (ti_s[i * TOPK + j], 0, 0)),
            pl.BlockSpec((1, MLP_DIM, DIM),
                         lambda i, j, ti_s, tp_s: (ti_s[i * TOPK + j], 0, 0)),
            pl.BlockSpec((1, 1, DIM),
                         lambda i, j, ti_s, tp_s: (ti_s[i * TOPK + j], 0, 0)),
        ],
        out_specs=pl.BlockSpec((1, N, DIM), lambda i, j, ti_s, tp_s: (i, 0, 0)),
    )
    out = pl.pallas_call(
        _moe_body,
        grid_spec=grid_spec,
        out_shape=jax.ShapeDtypeStruct((B, N, DIM), jnp.float32),
        compiler_params=pltpu.CompilerParams(
            dimension_semantics=("parallel", "arbitrary")),
    )(ti_flat, tp_flat, nx, xnew, W1.astype(jnp.bfloat16),
      b1.reshape(E, 1, MLP_DIM), W2.astype(jnp.bfloat16),
      b2.reshape(E, 1, DIM))

    return (out.reshape(B, H, W, DIM), all_probs, ti)


# speedup vs baseline: 1.1550x; 1.0854x over previous
import functools

import jax
import jax.numpy as jnp
from jax import lax
from jax.experimental import pallas as pl
from jax.experimental.pallas import tpu as pltpu

B, H, W = 32, 14, 14
DIM, HEADS, MLP_DIM = 384, 12, 1536
E, TOPK = 8, 2
HEAD_DIM = DIM // HEADS
N = H * W


IMGS = 4


def _attn_body(x_ref, g1_ref, be1_ref, Wq_ref, bq_ref, Wk_ref, bk_ref,
               Wv_ref, bv_ref, Wo_ref, bo_ref, g2_ref, be2_ref, Wg_ref,
               bg_ref, xnew_ref, nx_ref, ap_ref, ti_ref, tp_ref):
    for img in range(IMGS):
        _attn_one(img, x_ref, g1_ref, be1_ref, Wq_ref, bq_ref, Wk_ref,
                  bk_ref, Wv_ref, bv_ref, Wo_ref, bo_ref, g2_ref, be2_ref,
                  Wg_ref, bg_ref, xnew_ref, nx_ref, ap_ref, ti_ref, tp_ref)


def _attn_one(img, x_ref, g1_ref, be1_ref, Wq_ref, bq_ref, Wk_ref, bk_ref,
              Wv_ref, bv_ref, Wo_ref, bo_ref, g2_ref, be2_ref, Wg_ref,
              bg_ref, xnew_ref, nx_ref, ap_ref, ti_ref, tp_ref):
    xb = x_ref[img]

    mu = jnp.mean(xb, axis=-1, keepdims=True)
    var = jnp.mean((xb - mu) ** 2, axis=-1, keepdims=True)
    n1 = (xb - mu) / jnp.sqrt(var + 1e-5) * g1_ref[0] + be1_ref[0]

    q = jnp.dot(n1, Wq_ref[...], preferred_element_type=jnp.float32) + bq_ref[0]
    k = jnp.dot(n1, Wk_ref[...], preferred_element_type=jnp.float32) + bk_ref[0]
    v = jnp.dot(n1, Wv_ref[...], preferred_element_type=jnp.float32) + bv_ref[0]

    ri = lax.broadcasted_iota(jnp.int32, (N, N), 0) // W
    ci = lax.broadcasted_iota(jnp.int32, (N, N), 1) // W
    row_mask = ri == ci

    outs = []
    for h in range(HEADS):
        sl = slice(h * HEAD_DIM, (h + 1) * HEAD_DIM)
        s = lax.dot_general(q[:, sl], k[:, sl], (((1,), (1,)), ((), ())),
                            preferred_element_type=jnp.float32)
        s = s / (HEAD_DIM ** 0.5)
        s = jnp.where(row_mask, s, -1e30)
        m = jnp.max(s, axis=-1, keepdims=True)
        e = jnp.exp(s - m)
        p = e / jnp.sum(e, axis=-1, keepdims=True)
        outs.append(jnp.dot(p, v[:, sl], preferred_element_type=jnp.float32))
    o = jnp.concatenate(outs, axis=1)
    o = jnp.dot(o, Wo_ref[...], preferred_element_type=jnp.float32) + bo_ref[0]
    xn = xb + o
    xnew_ref[img] = xn

    mu2 = jnp.mean(xn, axis=-1, keepdims=True)
    var2 = jnp.mean((xn - mu2) ** 2, axis=-1, keepdims=True)
    nx = (xn - mu2) / jnp.sqrt(var2 + 1e-5) * g2_ref[0] + be2_ref[0]
    nx_ref[img] = nx

    pooled = jnp.mean(nx, axis=0, keepdims=True)
    logits = jnp.dot(pooled, Wg_ref[...],
                     preferred_element_type=jnp.float32) + bg_ref[0]
    lm = jnp.max(logits, axis=-1, keepdims=True)
    ex = jnp.exp(logits - lm)
    probs = ex / jnp.sum(ex, axis=-1, keepdims=True)
    ap_ref[img] = probs

    col = lax.broadcasted_iota(jnp.int32, (1, E), 1)
    m1 = jnp.max(probs, axis=-1, keepdims=True)
    i1 = jnp.min(jnp.where(probs == m1, col, E), axis=-1, keepdims=True)
    pm = jnp.where(col == i1, -1.0, probs)
    m2 = jnp.max(pm, axis=-1, keepdims=True)
    i2 = jnp.min(jnp.where(pm == m2, col, E), axis=-1, keepdims=True)
    den = m1 + m2 + 1e-8
    tp_ref[img] = jnp.concatenate([m1 / den, m2 / den], axis=1)
    ti_ref[img] = jnp.concatenate([i1, i2], axis=1)


def _moe_body(ti_ref, tp_ref, nx_ref, xnew_ref, W1_ref, b1_ref, W2_ref,
              b2_ref, out_ref):
    i = pl.program_id(0)
    j = pl.program_id(1)
    w = tp_ref[i * TOPK + j]
    h = jnp.dot(nx_ref[0].astype(jnp.bfloat16), W1_ref[0],
                preferred_element_type=jnp.float32) + b1_ref[0]
    h = 0.5 * h * (1.0 + lax.erf(h * (2.0 ** -0.5)))
    eo = jnp.dot(h.astype(jnp.bfloat16), W2_ref[0],
                 preferred_element_type=jnp.float32) + b2_ref[0]

    @pl.when(j == 0)
    def _():
        out_ref[0] = xnew_ref[0] + w * eo

    @pl.when(j == 1)
    def _():
        out_ref[0] = out_ref[0] + w * eo


def kernel(x, g1, be1, Wq, bq, Wk, bk, Wv, bv, Wo, bo, g2, be2, Wg, bg,
           W1, b1, W2, b2):
    xf = x.reshape(B, N, DIM)
    r2 = lambda a: a.reshape(1, -1)

    const2 = lambda shape: pl.BlockSpec(shape, lambda b: (0, 0))
    attn_out = pl.pallas_call(
        _attn_body,
        grid=(B // IMGS,),
        in_specs=[
            pl.BlockSpec((IMGS, N, DIM), lambda b: (b, 0, 0)),
            const2((1, DIM)), const2((1, DIM)),
            const2((DIM, DIM)), const2((1, DIM)),
            const2((DIM, DIM)), const2((1, DIM)),
            const2((DIM, DIM)), const2((1, DIM)),
            const2((DIM, DIM)), const2((1, DIM)),
            const2((1, DIM)), const2((1, DIM)),
            const2((DIM, E)), const2((1, E)),
        ],
        out_specs=[
            pl.BlockSpec((IMGS, N, DIM), lambda b: (b, 0, 0)),
            pl.BlockSpec((IMGS, N, DIM), lambda b: (b, 0, 0)),
            pl.BlockSpec((IMGS, 1, E), lambda b: (b, 0, 0)),
            pl.BlockSpec((IMGS, 1, TOPK), lambda b: (b, 0, 0)),
            pl.BlockSpec((IMGS, 1, TOPK), lambda b: (b, 0, 0)),
        ],
        out_shape=[
            jax.ShapeDtypeStruct((B, N, DIM), jnp.float32),
            jax.ShapeDtypeStruct((B, N, DIM), jnp.float32),
            jax.ShapeDtypeStruct((B, 1, E), jnp.float32),
            jax.ShapeDtypeStruct((B, 1, TOPK), jnp.int32),
            jax.ShapeDtypeStruct((B, 1, TOPK), jnp.float32),
        ],
        compiler_params=pltpu.CompilerParams(
            dimension_semantics=("parallel",)),
    )(xf, r2(g1), r2(be1), Wq, r2(bq), Wk, r2(bk), Wv, r2(bv), Wo, r2(bo),
      r2(g2), r2(be2), Wg, r2(bg))

    xnew, nx, ap3, ti3, tp3 = attn_out
    all_probs = ap3.reshape(B, E)
    ti = ti3.reshape(B, TOPK)
    ti_flat = ti.reshape(B * TOPK)
    tp_flat = tp3.reshape(B * TOPK)

    grid_spec = pltpu.PrefetchScalarGridSpec(
        num_scalar_prefetch=2,
        grid=(B, TOPK),
        in_specs=[
            pl.BlockSpec((1, N, DIM), lambda i, j, ti_s, tp_s: (i, 0, 0)),
            pl.BlockSpec((1, N, DIM), lambda i, j, ti_s, tp_s: (i, 0, 0)),
            pl.BlockSpec((1, DIM, MLP_DIM),
                         lambda i, j, ti_s, tp_s: (ti_s[i * TOPK + j], 0, 0)),
            pl.BlockSpec((1, 1, MLP_DIM),
                         lambda i, j, ti_s, tp_s: (ti_s[i * TOPK + j], 0, 0)),
            pl.BlockSpec((1, MLP_DIM, DIM),
                         lambda i, j, ti_s, tp_s: (ti_s[i * TOPK + j], 0, 0)),
            pl.BlockSpec((1, 1, DIM),
                         lambda i, j, ti_s, tp_s: (ti_s[i * TOPK + j], 0, 0)),
        ],
        out_specs=pl.BlockSpec((1, N, DIM), lambda i, j, ti_s, tp_s: (i, 0, 0)),
    )
    out = pl.pallas_call(
        _moe_body,
        grid_spec=grid_spec,
        out_shape=jax.ShapeDtypeStruct((B, N, DIM), jnp.float32),
        compiler_params=pltpu.CompilerParams(
            dimension_semantics=("parallel", "arbitrary")),
    )(ti_flat, tp_flat, nx, xnew, W1.astype(jnp.bfloat16),
      b1.reshape(E, 1, MLP_DIM), W2.astype(jnp.bfloat16),
      b2.reshape(E, 1, DIM))

    return (out.reshape(B, H, W, DIM), all_probs, ti)


# speedup vs baseline: 1.2163x; 1.0531x over previous
import functools

import jax
import jax.numpy as jnp
from jax import lax
from jax.experimental import pallas as pl
from jax.experimental.pallas import tpu as pltpu

B, H, W = 32, 14, 14
DIM, HEADS, MLP_DIM = 384, 12, 1536
E, TOPK = 8, 2
HEAD_DIM = DIM // HEADS
N = H * W


IMGS = 8


def _attn_body(x_ref, g1_ref, be1_ref, Wq_ref, bq_ref, Wk_ref, bk_ref,
               Wv_ref, bv_ref, Wo_ref, bo_ref, g2_ref, be2_ref, Wg_ref,
               bg_ref, xnew_ref, nx_ref, ap_ref, ti_ref, tp_ref):
    for img in range(IMGS):
        _attn_one(img, x_ref, g1_ref, be1_ref, Wq_ref, bq_ref, Wk_ref,
                  bk_ref, Wv_ref, bv_ref, Wo_ref, bo_ref, g2_ref, be2_ref,
                  Wg_ref, bg_ref, xnew_ref, nx_ref, ap_ref, ti_ref, tp_ref)


def _attn_one(img, x_ref, g1_ref, be1_ref, Wq_ref, bq_ref, Wk_ref, bk_ref,
              Wv_ref, bv_ref, Wo_ref, bo_ref, g2_ref, be2_ref, Wg_ref,
              bg_ref, xnew_ref, nx_ref, ap_ref, ti_ref, tp_ref):
    xb = x_ref[img]

    mu = jnp.mean(xb, axis=-1, keepdims=True)
    var = jnp.mean((xb - mu) ** 2, axis=-1, keepdims=True)
    n1 = (xb - mu) / jnp.sqrt(var + 1e-5) * g1_ref[0] + be1_ref[0]

    q = jnp.dot(n1, Wq_ref[...], preferred_element_type=jnp.float32) + bq_ref[0]
    k = jnp.dot(n1, Wk_ref[...], preferred_element_type=jnp.float32) + bk_ref[0]
    v = jnp.dot(n1, Wv_ref[...], preferred_element_type=jnp.float32) + bv_ref[0]

    ri = lax.broadcasted_iota(jnp.int32, (N, N), 0) // W
    ci = lax.broadcasted_iota(jnp.int32, (N, N), 1) // W
    row_mask = ri == ci

    outs = []
    for h in range(HEADS):
        sl = slice(h * HEAD_DIM, (h + 1) * HEAD_DIM)
        s = lax.dot_general(q[:, sl], k[:, sl], (((1,), (1,)), ((), ())),
                            preferred_element_type=jnp.float32)
        s = s / (HEAD_DIM ** 0.5)
        s = jnp.where(row_mask, s, -1e30)
        m = jnp.max(s, axis=-1, keepdims=True)
        e = jnp.exp(s - m)
        p = e / jnp.sum(e, axis=-1, keepdims=True)
        outs.append(jnp.dot(p, v[:, sl], preferred_element_type=jnp.float32))
    o = jnp.concatenate(outs, axis=1)
    o = jnp.dot(o, Wo_ref[...], preferred_element_type=jnp.float32) + bo_ref[0]
    xn = xb + o
    xnew_ref[img] = xn

    mu2 = jnp.mean(xn, axis=-1, keepdims=True)
    var2 = jnp.mean((xn - mu2) ** 2, axis=-1, keepdims=True)
    nx = (xn - mu2) / jnp.sqrt(var2 + 1e-5) * g2_ref[0] + be2_ref[0]
    nx_ref[img] = nx

    pooled = jnp.mean(nx, axis=0, keepdims=True)
    logits = jnp.dot(pooled, Wg_ref[...],
                     preferred_element_type=jnp.float32) + bg_ref[0]
    lm = jnp.max(logits, axis=-1, keepdims=True)
    ex = jnp.exp(logits - lm)
    probs = ex / jnp.sum(ex, axis=-1, keepdims=True)
    ap_ref[img] = probs

    col = lax.broadcasted_iota(jnp.int32, (1, E), 1)
    m1 = jnp.max(probs, axis=-1, keepdims=True)
    i1 = jnp.min(jnp.where(probs == m1, col, E), axis=-1, keepdims=True)
    pm = jnp.where(col == i1, -1.0, probs)
    m2 = jnp.max(pm, axis=-1, keepdims=True)
    i2 = jnp.min(jnp.where(pm == m2, col, E), axis=-1, keepdims=True)
    den = m1 + m2 + 1e-8
    tp_ref[img] = jnp.concatenate([m1 / den, m2 / den], axis=1)
    ti_ref[img] = jnp.concatenate([i1, i2], axis=1)


def _moe_body(ti_ref, tp_ref, nx_ref, xnew_ref, W1_ref, b1_ref, W2_ref,
              b2_ref, out_ref):
    i = pl.program_id(0)
    j = pl.program_id(1)
    w = tp_ref[i * TOPK + j]
    h = jnp.dot(nx_ref[0].astype(jnp.bfloat16), W1_ref[0],
                preferred_element_type=jnp.float32) + b1_ref[0]
    h = 0.5 * h * (1.0 + lax.erf(h * (2.0 ** -0.5)))
    eo = jnp.dot(h.astype(jnp.bfloat16), W2_ref[0],
                 preferred_element_type=jnp.float32) + b2_ref[0]

    @pl.when(j == 0)
    def _():
        out_ref[0] = xnew_ref[0] + w * eo

    @pl.when(j == 1)
    def _():
        out_ref[0] = out_ref[0] + w * eo


def kernel(x, g1, be1, Wq, bq, Wk, bk, Wv, bv, Wo, bo, g2, be2, Wg, bg,
           W1, b1, W2, b2):
    xf = x.reshape(B, N, DIM)
    r2 = lambda a: a.reshape(1, -1)

    const2 = lambda shape: pl.BlockSpec(shape, lambda b: (0, 0))
    attn_out = pl.pallas_call(
        _attn_body,
        grid=(B // IMGS,),
        in_specs=[
            pl.BlockSpec((IMGS, N, DIM), lambda b: (b, 0, 0)),
            const2((1, DIM)), const2((1, DIM)),
            const2((DIM, DIM)), const2((1, DIM)),
            const2((DIM, DIM)), const2((1, DIM)),
            const2((DIM, DIM)), const2((1, DIM)),
            const2((DIM, DIM)), const2((1, DIM)),
            const2((1, DIM)), const2((1, DIM)),
            const2((DIM, E)), const2((1, E)),
        ],
        out_specs=[
            pl.BlockSpec((IMGS, N, DIM), lambda b: (b, 0, 0)),
            pl.BlockSpec((IMGS, N, DIM), lambda b: (b, 0, 0)),
            pl.BlockSpec((IMGS, 1, E), lambda b: (b, 0, 0)),
            pl.BlockSpec((IMGS, 1, TOPK), lambda b: (b, 0, 0)),
            pl.BlockSpec((IMGS, 1, TOPK), lambda b: (b, 0, 0)),
        ],
        out_shape=[
            jax.ShapeDtypeStruct((B, N, DIM), jnp.float32),
            jax.ShapeDtypeStruct((B, N, DIM), jnp.float32),
            jax.ShapeDtypeStruct((B, 1, E), jnp.float32),
            jax.ShapeDtypeStruct((B, 1, TOPK), jnp.int32),
            jax.ShapeDtypeStruct((B, 1, TOPK), jnp.float32),
        ],
        compiler_params=pltpu.CompilerParams(
            dimension_semantics=("parallel",)),
    )(xf, r2(g1), r2(be1), Wq, r2(bq), Wk, r2(bk), Wv, r2(bv), Wo, r2(bo),
      r2(g2), r2(be2), Wg, r2(bg))

    xnew, nx, ap3, ti3, tp3 = attn_out
    all_probs = ap3.reshape(B, E)
    ti = ti3.reshape(B, TOPK)
    ti_flat = ti.reshape(B * TOPK)
    tp_flat = tp3.reshape(B * TOPK)

    grid_spec = pltpu.PrefetchScalarGridSpec(
        num_scalar_prefetch=2,
        grid=(B, TOPK),
        in_specs=[
            pl.BlockSpec((1, N, DIM), lambda i, j, ti_s, tp_s: (i, 0, 0)),
            pl.BlockSpec((1, N, DIM), lambda i, j, ti_s, tp_s: (i, 0, 0)),
            pl.BlockSpec((1, DIM, MLP_DIM),
                         lambda i, j, ti_s, tp_s: (ti_s[i * TOPK + j], 0, 0)),
            pl.BlockSpec((1, 1, MLP_DIM),
                         lambda i, j, ti_s, tp_s: (ti_s[i * TOPK + j], 0, 0)),
            pl.BlockSpec((1, MLP_DIM, DIM),
                         lambda i, j, ti_s, tp_s: (ti_s[i * TOPK + j], 0, 0)),
            pl.BlockSpec((1, 1, DIM),
                         lambda i, j, ti_s, tp_s: (ti_s[i * TOPK + j], 0, 0)),
        ],
        out_specs=pl.BlockSpec((1, N, DIM), lambda i, j, ti_s, tp_s: (i, 0, 0)),
    )
    out = pl.pallas_call(
        _moe_body,
        grid_spec=grid_spec,
        out_shape=jax.ShapeDtypeStruct((B, N, DIM), jnp.float32),
        compiler_params=pltpu.CompilerParams(
            dimension_semantics=("parallel", "arbitrary")),
    )(ti_flat, tp_flat, nx, xnew, W1.astype(jnp.bfloat16),
      b1.reshape(E, 1, MLP_DIM), W2.astype(jnp.bfloat16),
      b2.reshape(E, 1, DIM))

    return (out.reshape(B, H, W, DIM), all_probs, ti)
